# Initial kernel scaffold; baseline (speedup 1.0000x reference)
import jax
import jax.numpy as jnp
from jax.experimental import pallas as pl


def kernel(x, source, K0, K1, R, t, Wq, Wk, Wv, Wmerge, Wmlp1, Wmlp2, ln1_g, ln1_b, ln2_g, ln2_b):
    raise NotImplementedError("write your pallas kernel here")


# speedup vs baseline: 71.9341x; 71.9341x over previous
import jax
import jax.numpy as jnp
from jax.experimental import pallas as pl

_N = 1
_GH = 32
_GW = 32
_S = _GH * _GW
_D = 256
_NHEAD = 8
_DIM = _D // _NHEAD
_AW = 5
_C = max(_GH, _GW) * _AW
_SCALE = 16
_BQ = 128
_NBLK = _S // _BQ


def _proj_kernel(x_ref, s_ref, wq_ref, wk_ref, wv_ref, q_ref, k_ref, v_ref):
    x = x_ref[...]
    s = s_ref[...]
    q_ref[...] = jnp.dot(x, wq_ref[...], preferred_element_type=jnp.float32)
    k_ref[...] = jnp.dot(s, wk_ref[...], preferred_element_type=jnp.float32)
    v_ref[...] = jnp.dot(s, wv_ref[...], preferred_element_type=jnp.float32)


def _attn_kernel(ln_ref, q_ref, k_ref, v_ref, x_ref, wm_ref, a_ref, b_ref,
                 w2_ref, g1_ref, b1_ref, g2_ref, b2_ref, o_ref):
    lines = ln_ref[...]
    l0 = lines[:, 0:1]
    l1 = lines[:, 1:2]
    l2 = lines[:, 2:3]

    cols = jax.lax.broadcasted_iota(jnp.int32, (1, _S), 1)
    sx = (cols % _GW).astype(jnp.float32)
    sy = (cols // _GW).astype(jnp.float32)
    half = jnp.float32(_AW // 2)
    cy = -(l0 * sx + l2) / l1
    cx = -(l1 * sy + l2) / l0
    wy = (sy < cy + half) & (sy > cy - half)
    wx = (sx < cx + half) & (sx > cx - half)
    mode = jnp.abs(l1) > jnp.abs(l0)
    within = (mode & wy) | (jnp.logical_not(mode) & wx)
    cnt = jnp.sum(within.astype(jnp.int32), axis=1, keepdims=True)
    tail_kill = (cnt == 0) & (cols >= _C)

    q = q_ref[...]
    k = k_ref[...]
    v = v_ref[...]
    scale = jnp.float32(1.0 / (_DIM ** 0.5))
    neg = jnp.float32(-1e9)
    msg_parts = []
    for h in range(_NHEAD):
        qh = q[:, h * _DIM:(h + 1) * _DIM]
        kh = k[:, h * _DIM:(h + 1) * _DIM]
        vh = v[:, h * _DIM:(h + 1) * _DIM]
        lg = jax.lax.dot_general(qh, kh, (((1,), (1,)), ((), ())),
                                 preferred_element_type=jnp.float32) * scale
        lg = jnp.where(within, lg, neg)
        m = jnp.max(lg, axis=1, keepdims=True)
        e = jnp.exp(lg - m)
        e = jnp.where(tail_kill, jnp.float32(0.0), e)
        attn = e / jnp.sum(e, axis=1, keepdims=True)
        msg_parts.append(jnp.dot(attn, vh, preferred_element_type=jnp.float32))
    msg = jnp.concatenate(msg_parts, axis=1)

    merged = jnp.dot(msg, wm_ref[...], preferred_element_type=jnp.float32)
    mu = jnp.mean(merged, axis=1, keepdims=True)
    var = jnp.mean((merged - mu) ** 2, axis=1, keepdims=True)
    msgn = (merged - mu) / jnp.sqrt(var + 1e-5) * g1_ref[...] + b1_ref[...]

    xb = x_ref[...]
    h1 = (jnp.dot(xb, a_ref[...], preferred_element_type=jnp.float32)
          + jnp.dot(msgn, b_ref[...], preferred_element_type=jnp.float32))
    h1 = jnp.maximum(h1, jnp.float32(0.0))
    h2 = jnp.dot(h1, w2_ref[...], preferred_element_type=jnp.float32)
    mu2 = jnp.mean(h2, axis=1, keepdims=True)
    var2 = jnp.mean((h2 - mu2) ** 2, axis=1, keepdims=True)
    h2n = (h2 - mu2) / jnp.sqrt(var2 + 1e-5) * g2_ref[...] + b2_ref[...]
    o_ref[...] = xb + h2n


def kernel(x, source, K0, K1, R, t, Wq, Wk, Wv, Wmerge, Wmlp1, Wmlp2,
           ln1_g, ln1_b, ln2_g, ln2_b):
    K0s = jnp.concatenate([K0[:, :2, :] / _SCALE, K0[:, 2:, :]], axis=1)
    K1s = jnp.concatenate([K1[:, :2, :] / _SCALE, K1[:, 2:, :]], axis=1)
    tv = t[:, :, 0]
    t0, t1, t2 = tv[:, 0], tv[:, 1], tv[:, 2]
    z = jnp.zeros_like(t0)
    skew = jnp.stack([
        jnp.stack([z, -t2, t1], axis=-1),
        jnp.stack([t2, z, -t0], axis=-1),
        jnp.stack([-t1, t0, z], axis=-1)], axis=1)
    F = jnp.transpose(jnp.linalg.inv(K1s), (0, 2, 1)) @ skew @ R @ jnp.linalg.inv(K0s)
    xs = jnp.arange(_GW, dtype=jnp.float32)
    ys = jnp.arange(_GH, dtype=jnp.float32)
    gx, gy = jnp.meshgrid(xs, ys, indexing='xy')
    coord = jnp.stack([gx, gy], axis=-1).reshape(_S, 2)
    p0 = jnp.concatenate([coord, jnp.ones((_S, 1), dtype=jnp.float32)], axis=-1)
    lines = jnp.einsum('nij,sj->nsi', F, p0).reshape(_S, 3)

    idx = jnp.arange(_D)
    src = (idx % _DIM) * _NHEAD + idx // _DIM
    wq = jnp.transpose(Wq[src, :])
    wk = jnp.transpose(Wk[src, :])
    wv = jnp.transpose(Wv[src, :])
    wm = jnp.transpose(Wmerge)[src, :]
    a = jnp.transpose(Wmlp1[:, :_D])
    b = jnp.transpose(Wmlp1[:, _D:])
    w2 = jnp.transpose(Wmlp2)
    g1 = ln1_g.reshape(1, _D)
    b1 = ln1_b.reshape(1, _D)
    g2 = ln2_g.reshape(1, _D)
    b2 = ln2_b.reshape(1, _D)

    x2 = x.reshape(_S, _D)
    s2 = source.reshape(_S, _D)

    q, k, v = pl.pallas_call(
        _proj_kernel,
        out_shape=[jax.ShapeDtypeStruct((_S, _D), jnp.float32)] * 3,
    )(x2, s2, wq, wk, wv)

    out = pl.pallas_call(
        _attn_kernel,
        grid=(_NBLK,),
        in_specs=[
            pl.BlockSpec((_BQ, 3), lambda i: (i, 0)),
            pl.BlockSpec((_BQ, _D), lambda i: (i, 0)),
            pl.BlockSpec((_S, _D), lambda i: (0, 0)),
            pl.BlockSpec((_S, _D), lambda i: (0, 0)),
            pl.BlockSpec((_BQ, _D), lambda i: (i, 0)),
            pl.BlockSpec((_D, _D), lambda i: (0, 0)),
            pl.BlockSpec((_D, 2 * _D), lambda i: (0, 0)),
            pl.BlockSpec((_D, 2 * _D), lambda i: (0, 0)),
            pl.BlockSpec((2 * _D, _D), lambda i: (0, 0)),
            pl.BlockSpec((1, _D), lambda i: (0, 0)),
            pl.BlockSpec((1, _D), lambda i: (0, 0)),
            pl.BlockSpec((1, _D), lambda i: (0, 0)),
            pl.BlockSpec((1, _D), lambda i: (0, 0)),
        ],
        out_specs=pl.BlockSpec((_BQ, _D), lambda i: (i, 0)),
        out_shape=jax.ShapeDtypeStruct((_S, _D), jnp.float32),
    )(lines, q, k, v, x2, wm, a, b, w2, g1, b1, g2, b2)

    return out.reshape(_N, _S, _D)


# speedup vs baseline: 80.2700x; 1.1159x over previous
import jax
import jax.numpy as jnp
from jax.experimental import pallas as pl

_N = 1
_GH = 32
_GW = 32
_S = _GH * _GW
_D = 256
_NHEAD = 8
_DIM = _D // _NHEAD
_AW = 5
_C = max(_GH, _GW) * _AW
_SCALE = 16
_BQ = 128
_NBLK = _S // _BQ

_NT = (((1,), (1,)), ((), ()))


def _perm_matrix():
    row = jax.lax.broadcasted_iota(jnp.int32, (_D, _D), 0)
    col = jax.lax.broadcasted_iota(jnp.int32, (_D, _D), 1)
    tgt = (col % _DIM) * _NHEAD + col // _DIM
    return (row == tgt).astype(jnp.float32)


def _proj_kernel(x_ref, s_ref, wq_ref, wk_ref, wv_ref, q_ref, k_ref, v_ref):
    p = _perm_matrix()
    x = x_ref[...]
    s = s_ref[...]
    f32 = jnp.float32
    q = jax.lax.dot_general(x, wq_ref[...], _NT, preferred_element_type=f32)
    k = jax.lax.dot_general(s, wk_ref[...], _NT, preferred_element_type=f32)
    v = jax.lax.dot_general(s, wv_ref[...], _NT, preferred_element_type=f32)
    q_ref[...] = jnp.dot(q, p, preferred_element_type=f32)
    k_ref[...] = jnp.dot(k, p, preferred_element_type=f32)
    v_ref[...] = jnp.dot(v, p, preferred_element_type=f32)


def _attn_kernel(ln_ref, q_ref, k_ref, v_ref, x_ref, wm_ref, w1_ref,
                 w2_ref, g1_ref, b1_ref, g2_ref, b2_ref, o_ref):
    lines = ln_ref[...]
    l0 = lines[:, 0:1]
    l1 = lines[:, 1:2]
    l2 = lines[:, 2:3]

    cols = jax.lax.broadcasted_iota(jnp.int32, (1, _S), 1)
    sx = (cols % _GW).astype(jnp.float32)
    sy = (cols // _GW).astype(jnp.float32)
    half = jnp.float32(_AW // 2)
    cy = -(l0 * sx + l2) / l1
    cx = -(l1 * sy + l2) / l0
    wy = (sy < cy + half) & (sy > cy - half)
    wx = (sx < cx + half) & (sx > cx - half)
    mode = jnp.abs(l1) > jnp.abs(l0)
    within = (mode & wy) | (jnp.logical_not(mode) & wx)
    cnt = jnp.sum(within.astype(jnp.int32), axis=1, keepdims=True)
    tail_kill = (cnt == 0) & (cols >= _C)

    q = q_ref[...]
    k = k_ref[...]
    v = v_ref[...]
    f32 = jnp.float32
    scale = jnp.float32(1.0 / (_DIM ** 0.5))
    neg = jnp.float32(-1e9)
    msg_parts = []
    for h in range(_NHEAD):
        qh = q[:, h * _DIM:(h + 1) * _DIM]
        kh = k[:, h * _DIM:(h + 1) * _DIM]
        vh = v[:, h * _DIM:(h + 1) * _DIM]
        lg = jax.lax.dot_general(qh, kh, _NT, preferred_element_type=f32) * scale
        lg = jnp.where(within, lg, neg)
        m = jnp.max(lg, axis=1, keepdims=True)
        e = jnp.exp(lg - m)
        e = jnp.where(tail_kill, jnp.float32(0.0), e)
        attn = e / jnp.sum(e, axis=1, keepdims=True)
        msg_parts.append(jnp.dot(attn, vh, preferred_element_type=f32))
    msg = jnp.concatenate(msg_parts, axis=1)

    pt = jnp.transpose(_perm_matrix())
    msg_ref_order = jnp.dot(msg, pt, preferred_element_type=f32)
    merged = jax.lax.dot_general(msg_ref_order, wm_ref[...], _NT,
                                 preferred_element_type=f32)
    mu = jnp.mean(merged, axis=1, keepdims=True)
    var = jnp.mean((merged - mu) ** 2, axis=1, keepdims=True)
    msgn = (merged - mu) / jnp.sqrt(var + 1e-5) * g1_ref[...] + b1_ref[...]

    xb = x_ref[...]
    w1 = w1_ref[...]
    h1 = (jax.lax.dot_general(xb, w1[:, :_D], _NT, preferred_element_type=f32)
          + jax.lax.dot_general(msgn, w1[:, _D:], _NT, preferred_element_type=f32))
    h1 = jnp.maximum(h1, jnp.float32(0.0))
    h2 = jax.lax.dot_general(h1, w2_ref[...], _NT, preferred_element_type=f32)
    mu2 = jnp.mean(h2, axis=1, keepdims=True)
    var2 = jnp.mean((h2 - mu2) ** 2, axis=1, keepdims=True)
    h2n = (h2 - mu2) / jnp.sqrt(var2 + 1e-5) * g2_ref[...] + b2_ref[...]
    o_ref[...] = xb + h2n


def kernel(x, source, K0, K1, R, t, Wq, Wk, Wv, Wmerge, Wmlp1, Wmlp2,
           ln1_g, ln1_b, ln2_g, ln2_b):
    K0s = jnp.concatenate([K0[:, :2, :] / _SCALE, K0[:, 2:, :]], axis=1)
    K1s = jnp.concatenate([K1[:, :2, :] / _SCALE, K1[:, 2:, :]], axis=1)
    tv = t[:, :, 0]
    t0, t1, t2 = tv[:, 0], tv[:, 1], tv[:, 2]
    z = jnp.zeros_like(t0)
    skew = jnp.stack([
        jnp.stack([z, -t2, t1], axis=-1),
        jnp.stack([t2, z, -t0], axis=-1),
        jnp.stack([-t1, t0, z], axis=-1)], axis=1)
    F = jnp.transpose(jnp.linalg.inv(K1s), (0, 2, 1)) @ skew @ R @ jnp.linalg.inv(K0s)
    xs = jnp.arange(_GW, dtype=jnp.float32)
    ys = jnp.arange(_GH, dtype=jnp.float32)
    gx, gy = jnp.meshgrid(xs, ys, indexing='xy')
    coord = jnp.stack([gx, gy], axis=-1).reshape(_S, 2)
    p0 = jnp.concatenate([coord, jnp.ones((_S, 1), dtype=jnp.float32)], axis=-1)
    lines = jnp.einsum('nij,sj->nsi', F, p0).reshape(_S, 3)

    g1 = ln1_g.reshape(1, _D)
    b1 = ln1_b.reshape(1, _D)
    g2 = ln2_g.reshape(1, _D)
    b2 = ln2_b.reshape(1, _D)
    x2 = x.reshape(_S, _D)
    s2 = source.reshape(_S, _D)

    q, k, v = pl.pallas_call(
        _proj_kernel,
        out_shape=[jax.ShapeDtypeStruct((_S, _D), jnp.float32)] * 3,
    )(x2, s2, Wq, Wk, Wv)

    out = pl.pallas_call(
        _attn_kernel,
        grid=(_NBLK,),
        in_specs=[
            pl.BlockSpec((_BQ, 3), lambda i: (i, 0)),
            pl.BlockSpec((_BQ, _D), lambda i: (i, 0)),
            pl.BlockSpec((_S, _D), lambda i: (0, 0)),
            pl.BlockSpec((_S, _D), lambda i: (0, 0)),
            pl.BlockSpec((_BQ, _D), lambda i: (i, 0)),
            pl.BlockSpec((_D, _D), lambda i: (0, 0)),
            pl.BlockSpec((2 * _D, 2 * _D), lambda i: (0, 0)),
            pl.BlockSpec((_D, 2 * _D), lambda i: (0, 0)),
            pl.BlockSpec((1, _D), lambda i: (0, 0)),
            pl.BlockSpec((1, _D), lambda i: (0, 0)),
            pl.BlockSpec((1, _D), lambda i: (0, 0)),
            pl.BlockSpec((1, _D), lambda i: (0, 0)),
        ],
        out_specs=pl.BlockSpec((_BQ, _D), lambda i: (i, 0)),
        out_shape=jax.ShapeDtypeStruct((_S, _D), jnp.float32),
    )(lines, q, k, v, x2, Wmerge, Wmlp1, Wmlp2, g1, b1, g2, b2)

    return out.reshape(_N, _S, _D)


# speedup vs baseline: 84.3520x; 1.0509x over previous
import jax
import jax.numpy as jnp
from jax.experimental import pallas as pl
from jax.experimental.pallas import tpu as pltpu

_N = 1
_GH = 32
_GW = 32
_S = _GH * _GW
_D = 256
_NHEAD = 8
_DIM = _D // _NHEAD
_AW = 5
_C = max(_GH, _GW) * _AW
_SCALE = 16
_BQ = 128
_NBLK = _S // _BQ

_NT = (((1,), (1,)), ((), ()))


def _perm_matrix():
    row = jax.lax.broadcasted_iota(jnp.int32, (_D, _D), 0)
    col = jax.lax.broadcasted_iota(jnp.int32, (_D, _D), 1)
    tgt = (col % _DIM) * _NHEAD + col // _DIM
    return (row == tgt).astype(jnp.float32)


def _attn_kernel(ln_ref, x_ref, s_ref, wq_ref, wk_ref, wv_ref, wm_ref, w1_ref,
                 w2_ref, g1_ref, b1_ref, g2_ref, b2_ref, o_ref, k_s, v_s):
    f32 = jnp.float32
    p = _perm_matrix()

    @pl.when(pl.program_id(0) == 0)
    def _project_kv():
        s = s_ref[...]
        kr = jax.lax.dot_general(s, wk_ref[...], _NT, preferred_element_type=f32)
        vr = jax.lax.dot_general(s, wv_ref[...], _NT, preferred_element_type=f32)
        k_s[...] = jnp.dot(kr, p, preferred_element_type=f32)
        v_s[...] = jnp.dot(vr, p, preferred_element_type=f32)

    lines = ln_ref[...]
    l0 = lines[:, 0:1]
    l1 = lines[:, 1:2]
    l2 = lines[:, 2:3]

    cols = jax.lax.broadcasted_iota(jnp.int32, (1, _S), 1)
    sx = (cols % _GW).astype(jnp.float32)
    sy = (cols // _GW).astype(jnp.float32)
    half = jnp.float32(_AW // 2)
    cy = -(l0 * sx + l2) / l1
    cx = -(l1 * sy + l2) / l0
    wy = (sy < cy + half) & (sy > cy - half)
    wx = (sx < cx + half) & (sx > cx - half)
    mode = jnp.abs(l1) > jnp.abs(l0)
    within = (mode & wy) | (jnp.logical_not(mode) & wx)
    cnt = jnp.sum(within.astype(jnp.int32), axis=1, keepdims=True)
    tail_kill = (cnt == 0) & (cols >= _C)

    xb = x_ref[...]
    qr = jax.lax.dot_general(xb, wq_ref[...], _NT, preferred_element_type=f32)
    q = jnp.dot(qr, p, preferred_element_type=f32)
    k = k_s[...]
    v = v_s[...]
    scale = jnp.float32(1.0 / (_DIM ** 0.5))
    neg = jnp.float32(-1e9)
    msg_parts = []
    for h in range(_NHEAD):
        qh = q[:, h * _DIM:(h + 1) * _DIM]
        kh = k[:, h * _DIM:(h + 1) * _DIM]
        vh = v[:, h * _DIM:(h + 1) * _DIM]
        lg = jax.lax.dot_general(qh, kh, _NT, preferred_element_type=f32) * scale
        lg = jnp.where(within, lg, neg)
        m = jnp.max(lg, axis=1, keepdims=True)
        e = jnp.exp(lg - m)
        e = jnp.where(tail_kill, jnp.float32(0.0), e)
        attn = e / jnp.sum(e, axis=1, keepdims=True)
        msg_parts.append(jnp.dot(attn, vh, preferred_element_type=f32))
    msg = jnp.concatenate(msg_parts, axis=1)

    pt = jnp.transpose(_perm_matrix())
    msg_ref_order = jnp.dot(msg, pt, preferred_element_type=f32)
    merged = jax.lax.dot_general(msg_ref_order, wm_ref[...], _NT,
                                 preferred_element_type=f32)
    mu = jnp.mean(merged, axis=1, keepdims=True)
    var = jnp.mean((merged - mu) ** 2, axis=1, keepdims=True)
    msgn = (merged - mu) / jnp.sqrt(var + 1e-5) * g1_ref[...] + b1_ref[...]

    w1 = w1_ref[...]
    h1 = (jax.lax.dot_general(xb, w1[:, :_D], _NT, preferred_element_type=f32)
          + jax.lax.dot_general(msgn, w1[:, _D:], _NT, preferred_element_type=f32))
    h1 = jnp.maximum(h1, jnp.float32(0.0))
    h2 = jax.lax.dot_general(h1, w2_ref[...], _NT, preferred_element_type=f32)
    mu2 = jnp.mean(h2, axis=1, keepdims=True)
    var2 = jnp.mean((h2 - mu2) ** 2, axis=1, keepdims=True)
    h2n = (h2 - mu2) / jnp.sqrt(var2 + 1e-5) * g2_ref[...] + b2_ref[...]
    o_ref[...] = xb + h2n


def kernel(x, source, K0, K1, R, t, Wq, Wk, Wv, Wmerge, Wmlp1, Wmlp2,
           ln1_g, ln1_b, ln2_g, ln2_b):
    K0s = jnp.concatenate([K0[:, :2, :] / _SCALE, K0[:, 2:, :]], axis=1)
    K1s = jnp.concatenate([K1[:, :2, :] / _SCALE, K1[:, 2:, :]], axis=1)
    tv = t[:, :, 0]
    t0, t1, t2 = tv[:, 0], tv[:, 1], tv[:, 2]
    z = jnp.zeros_like(t0)
    skew = jnp.stack([
        jnp.stack([z, -t2, t1], axis=-1),
        jnp.stack([t2, z, -t0], axis=-1),
        jnp.stack([-t1, t0, z], axis=-1)], axis=1)
    F = jnp.transpose(jnp.linalg.inv(K1s), (0, 2, 1)) @ skew @ R @ jnp.linalg.inv(K0s)
    xs = jnp.arange(_GW, dtype=jnp.float32)
    ys = jnp.arange(_GH, dtype=jnp.float32)
    gx, gy = jnp.meshgrid(xs, ys, indexing='xy')
    coord = jnp.stack([gx, gy], axis=-1).reshape(_S, 2)
    p0 = jnp.concatenate([coord, jnp.ones((_S, 1), dtype=jnp.float32)], axis=-1)
    lines = jnp.einsum('nij,sj->nsi', F, p0).reshape(_S, 3)

    g1 = ln1_g.reshape(1, _D)
    b1 = ln1_b.reshape(1, _D)
    g2 = ln2_g.reshape(1, _D)
    b2 = ln2_b.reshape(1, _D)
    x2 = x.reshape(_S, _D)
    s2 = source.reshape(_S, _D)

    out = pl.pallas_call(
        _attn_kernel,
        grid=(_NBLK,),
        in_specs=[
            pl.BlockSpec((_BQ, 3), lambda i: (i, 0)),
            pl.BlockSpec((_BQ, _D), lambda i: (i, 0)),
            pl.BlockSpec((_S, _D), lambda i: (0, 0)),
            pl.BlockSpec((_D, _D), lambda i: (0, 0)),
            pl.BlockSpec((_D, _D), lambda i: (0, 0)),
            pl.BlockSpec((_D, _D), lambda i: (0, 0)),
            pl.BlockSpec((_D, _D), lambda i: (0, 0)),
            pl.BlockSpec((2 * _D, 2 * _D), lambda i: (0, 0)),
            pl.BlockSpec((_D, 2 * _D), lambda i: (0, 0)),
            pl.BlockSpec((1, _D), lambda i: (0, 0)),
            pl.BlockSpec((1, _D), lambda i: (0, 0)),
            pl.BlockSpec((1, _D), lambda i: (0, 0)),
            pl.BlockSpec((1, _D), lambda i: (0, 0)),
        ],
        out_specs=pl.BlockSpec((_BQ, _D), lambda i: (i, 0)),
        out_shape=jax.ShapeDtypeStruct((_S, _D), jnp.float32),
        scratch_shapes=[
            pltpu.VMEM((_S, _D), jnp.float32),
            pltpu.VMEM((_S, _D), jnp.float32),
        ],
    )(lines, x2, s2, Wq, Wk, Wv, Wmerge, Wmlp1, Wmlp2, g1, b1, g2, b2)

    return out.reshape(_N, _S, _D)


# speedup vs baseline: 114.1454x; 1.3532x over previous
import jax
import jax.numpy as jnp
from jax.experimental import pallas as pl
from jax.experimental.pallas import tpu as pltpu

_N = 1
_GH = 32
_GW = 32
_S = _GH * _GW
_D = 256
_NHEAD = 8
_DIM = _D // _NHEAD
_AW = 5
_C = max(_GH, _GW) * _AW
_SCALE = 16
_BQ = 256
_NBLK = _S // _BQ

_NT = (((1,), (1,)), ((), ()))


def _perm_matrix():
    row = jax.lax.broadcasted_iota(jnp.int32, (_D, _D), 0)
    col = jax.lax.broadcasted_iota(jnp.int32, (_D, _D), 1)
    tgt = (col % _DIM) * _NHEAD + col // _DIM
    return (row == tgt).astype(jnp.float32)


def _attn_kernel(ln_ref, x_ref, s_ref, wq_ref, wk_ref, wv_ref, wm_ref, w1_ref,
                 w2_ref, g1_ref, b1_ref, g2_ref, b2_ref, o_ref, k_s, v_s):
    f32 = jnp.float32
    p = _perm_matrix()

    @pl.when(pl.program_id(0) == 0)
    def _project_kv():
        s = s_ref[...]
        kr = jax.lax.dot_general(s, wk_ref[...], _NT, preferred_element_type=f32)
        vr = jax.lax.dot_general(s, wv_ref[...], _NT, preferred_element_type=f32)
        k_s[...] = jnp.dot(kr, p, preferred_element_type=f32)
        v_s[...] = jnp.dot(vr, p, preferred_element_type=f32)

    lines = ln_ref[...]
    l0 = lines[:, 0:1]
    l1 = lines[:, 1:2]
    l2 = lines[:, 2:3]

    cols = jax.lax.broadcasted_iota(jnp.int32, (1, _S), 1)
    sx = (cols % _GW).astype(jnp.float32)
    sy = (cols // _GW).astype(jnp.float32)
    half = jnp.float32(_AW // 2)
    cy = -(l0 * sx + l2) / l1
    cx = -(l1 * sy + l2) / l0
    wy = (sy < cy + half) & (sy > cy - half)
    wx = (sx < cx + half) & (sx > cx - half)
    mode = jnp.abs(l1) > jnp.abs(l0)
    within = (mode & wy) | (jnp.logical_not(mode) & wx)
    cnt = jnp.sum(within.astype(jnp.int32), axis=1, keepdims=True)
    fallback = (cnt == 0) & (cols < _C)

    xb = x_ref[...]
    qr = jax.lax.dot_general(xb, wq_ref[...], _NT, preferred_element_type=f32)
    q = jnp.dot(qr, p, preferred_element_type=f32)
    k = k_s[...]
    v = v_s[...]
    scale = jnp.float32(1.0 / (_DIM ** 0.5))
    msg_parts = []
    for h in range(_NHEAD):
        qh = q[:, h * _DIM:(h + 1) * _DIM]
        kh = k[:, h * _DIM:(h + 1) * _DIM]
        vh = v[:, h * _DIM:(h + 1) * _DIM]
        lg = jax.lax.dot_general(qh, kh, _NT, preferred_element_type=f32) * scale
        e = jnp.where(within, jnp.exp(lg), jnp.float32(0.0))
        e = jnp.where(fallback, jnp.float32(1.0), e)
        attn = e / jnp.sum(e, axis=1, keepdims=True)
        msg_parts.append(jnp.dot(attn, vh, preferred_element_type=f32))
    msg = jnp.concatenate(msg_parts, axis=1)

    pt = jnp.transpose(_perm_matrix())
    msg_ref_order = jnp.dot(msg, pt, preferred_element_type=f32)
    merged = jax.lax.dot_general(msg_ref_order, wm_ref[...], _NT,
                                 preferred_element_type=f32)
    mu = jnp.mean(merged, axis=1, keepdims=True)
    var = jnp.mean((merged - mu) ** 2, axis=1, keepdims=True)
    msgn = (merged - mu) / jnp.sqrt(var + 1e-5) * g1_ref[...] + b1_ref[...]

    w1 = w1_ref[...]
    h1 = (jax.lax.dot_general(xb, w1[:, :_D], _NT, preferred_element_type=f32)
          + jax.lax.dot_general(msgn, w1[:, _D:], _NT, preferred_element_type=f32))
    h1 = jnp.maximum(h1, jnp.float32(0.0))
    h2 = jax.lax.dot_general(h1, w2_ref[...], _NT, preferred_element_type=f32)
    mu2 = jnp.mean(h2, axis=1, keepdims=True)
    var2 = jnp.mean((h2 - mu2) ** 2, axis=1, keepdims=True)
    h2n = (h2 - mu2) / jnp.sqrt(var2 + 1e-5) * g2_ref[...] + b2_ref[...]
    o_ref[...] = xb + h2n


def kernel(x, source, K0, K1, R, t, Wq, Wk, Wv, Wmerge, Wmlp1, Wmlp2,
           ln1_g, ln1_b, ln2_g, ln2_b):
    K0s = jnp.concatenate([K0[:, :2, :] / _SCALE, K0[:, 2:, :]], axis=1)
    K1s = jnp.concatenate([K1[:, :2, :] / _SCALE, K1[:, 2:, :]], axis=1)
    tv = t[:, :, 0]
    t0, t1, t2 = tv[:, 0], tv[:, 1], tv[:, 2]
    z = jnp.zeros_like(t0)
    skew = jnp.stack([
        jnp.stack([z, -t2, t1], axis=-1),
        jnp.stack([t2, z, -t0], axis=-1),
        jnp.stack([-t1, t0, z], axis=-1)], axis=1)
    F = jnp.transpose(jnp.linalg.inv(K1s), (0, 2, 1)) @ skew @ R @ jnp.linalg.inv(K0s)
    xs = jnp.arange(_GW, dtype=jnp.float32)
    ys = jnp.arange(_GH, dtype=jnp.float32)
    gx, gy = jnp.meshgrid(xs, ys, indexing='xy')
    coord = jnp.stack([gx, gy], axis=-1).reshape(_S, 2)
    p0 = jnp.concatenate([coord, jnp.ones((_S, 1), dtype=jnp.float32)], axis=-1)
    lines = jnp.einsum('nij,sj->nsi', F, p0).reshape(_S, 3)

    g1 = ln1_g.reshape(1, _D)
    b1 = ln1_b.reshape(1, _D)
    g2 = ln2_g.reshape(1, _D)
    b2 = ln2_b.reshape(1, _D)
    x2 = x.reshape(_S, _D)
    s2 = source.reshape(_S, _D)

    out = pl.pallas_call(
        _attn_kernel,
        grid=(_NBLK,),
        in_specs=[
            pl.BlockSpec((_BQ, 3), lambda i: (i, 0)),
            pl.BlockSpec((_BQ, _D), lambda i: (i, 0)),
            pl.BlockSpec((_S, _D), lambda i: (0, 0)),
            pl.BlockSpec((_D, _D), lambda i: (0, 0)),
            pl.BlockSpec((_D, _D), lambda i: (0, 0)),
            pl.BlockSpec((_D, _D), lambda i: (0, 0)),
            pl.BlockSpec((_D, _D), lambda i: (0, 0)),
            pl.BlockSpec((2 * _D, 2 * _D), lambda i: (0, 0)),
            pl.BlockSpec((_D, 2 * _D), lambda i: (0, 0)),
            pl.BlockSpec((1, _D), lambda i: (0, 0)),
            pl.BlockSpec((1, _D), lambda i: (0, 0)),
            pl.BlockSpec((1, _D), lambda i: (0, 0)),
            pl.BlockSpec((1, _D), lambda i: (0, 0)),
        ],
        out_specs=pl.BlockSpec((_BQ, _D), lambda i: (i, 0)),
        out_shape=jax.ShapeDtypeStruct((_S, _D), jnp.float32),
        scratch_shapes=[
            pltpu.VMEM((_S, _D), jnp.float32),
            pltpu.VMEM((_S, _D), jnp.float32),
        ],
    )(lines, x2, s2, Wq, Wk, Wv, Wmerge, Wmlp1, Wmlp2, g1, b1, g2, b2)

    return out.reshape(_N, _S, _D)


# speedup vs baseline: 118.5048x; 1.0382x over previous
import jax
import jax.numpy as jnp
from jax.experimental import pallas as pl
from jax.experimental.pallas import tpu as pltpu

_N = 1
_GH = 32
_GW = 32
_S = _GH * _GW
_D = 256
_NHEAD = 8
_DIM = _D // _NHEAD
_AW = 5
_C = max(_GH, _GW) * _AW
_SCALE = 16
_BQ = 1024
_NBLK = _S // _BQ

_NT = (((1,), (1,)), ((), ()))


def _perm_matrix():
    row = jax.lax.broadcasted_iota(jnp.int32, (_D, _D), 0)
    col = jax.lax.broadcasted_iota(jnp.int32, (_D, _D), 1)
    tgt = (col % _DIM) * _NHEAD + col // _DIM
    return (row == tgt).astype(jnp.float32)


def _attn_kernel(ln_ref, x_ref, s_ref, wq_ref, wk_ref, wv_ref, wm_ref, w1_ref,
                 w2_ref, g1_ref, b1_ref, g2_ref, b2_ref, o_ref, k_s, v_s):
    f32 = jnp.float32
    p = _perm_matrix()

    @pl.when(pl.program_id(0) == 0)
    def _project_kv():
        s = s_ref[...]
        kr = jax.lax.dot_general(s, wk_ref[...], _NT, preferred_element_type=f32)
        vr = jax.lax.dot_general(s, wv_ref[...], _NT, preferred_element_type=f32)
        k_s[...] = jnp.dot(kr, p, preferred_element_type=f32)
        v_s[...] = jnp.dot(vr, p, preferred_element_type=f32)

    lines = ln_ref[...]
    l0 = lines[:, 0:1]
    l1 = lines[:, 1:2]
    l2 = lines[:, 2:3]

    cols = jax.lax.broadcasted_iota(jnp.int32, (1, _S), 1)
    sx = (cols % _GW).astype(jnp.float32)
    sy = (cols // _GW).astype(jnp.float32)
    half = jnp.float32(_AW // 2)
    cy = -(l0 * sx + l2) / l1
    cx = -(l1 * sy + l2) / l0
    wy = (sy < cy + half) & (sy > cy - half)
    wx = (sx < cx + half) & (sx > cx - half)
    mode = jnp.abs(l1) > jnp.abs(l0)
    within = (mode & wy) | (jnp.logical_not(mode) & wx)
    cnt = jnp.sum(within.astype(jnp.int32), axis=1, keepdims=True)
    fallback = (cnt == 0) & (cols < _C)

    xb = x_ref[...]
    qr = jax.lax.dot_general(xb, wq_ref[...], _NT, preferred_element_type=f32)
    q = jnp.dot(qr, p, preferred_element_type=f32)
    k = k_s[...]
    v = v_s[...]
    scale = jnp.float32(1.0 / (_DIM ** 0.5))
    msg_parts = []
    for h in range(_NHEAD):
        qh = q[:, h * _DIM:(h + 1) * _DIM]
        kh = k[:, h * _DIM:(h + 1) * _DIM]
        vh = v[:, h * _DIM:(h + 1) * _DIM]
        lg = jax.lax.dot_general(qh, kh, _NT, preferred_element_type=f32) * scale
        e = jnp.where(within, jnp.exp(lg), jnp.float32(0.0))
        e = jnp.where(fallback, jnp.float32(1.0), e)
        attn = e / jnp.sum(e, axis=1, keepdims=True)
        msg_parts.append(jnp.dot(attn, vh, preferred_element_type=f32))
    msg = jnp.concatenate(msg_parts, axis=1)

    pt = jnp.transpose(_perm_matrix())
    msg_ref_order = jnp.dot(msg, pt, preferred_element_type=f32)
    merged = jax.lax.dot_general(msg_ref_order, wm_ref[...], _NT,
                                 preferred_element_type=f32)
    mu = jnp.mean(merged, axis=1, keepdims=True)
    var = jnp.mean((merged - mu) ** 2, axis=1, keepdims=True)
    msgn = (merged - mu) / jnp.sqrt(var + 1e-5) * g1_ref[...] + b1_ref[...]

    w1 = w1_ref[...]
    h1 = (jax.lax.dot_general(xb, w1[:, :_D], _NT, preferred_element_type=f32)
          + jax.lax.dot_general(msgn, w1[:, _D:], _NT, preferred_element_type=f32))
    h1 = jnp.maximum(h1, jnp.float32(0.0))
    h2 = jax.lax.dot_general(h1, w2_ref[...], _NT, preferred_element_type=f32)
    mu2 = jnp.mean(h2, axis=1, keepdims=True)
    var2 = jnp.mean((h2 - mu2) ** 2, axis=1, keepdims=True)
    h2n = (h2 - mu2) / jnp.sqrt(var2 + 1e-5) * g2_ref[...] + b2_ref[...]
    o_ref[...] = xb + h2n


def kernel(x, source, K0, K1, R, t, Wq, Wk, Wv, Wmerge, Wmlp1, Wmlp2,
           ln1_g, ln1_b, ln2_g, ln2_b):
    K0s = jnp.concatenate([K0[:, :2, :] / _SCALE, K0[:, 2:, :]], axis=1)
    K1s = jnp.concatenate([K1[:, :2, :] / _SCALE, K1[:, 2:, :]], axis=1)
    tv = t[:, :, 0]
    t0, t1, t2 = tv[:, 0], tv[:, 1], tv[:, 2]
    z = jnp.zeros_like(t0)
    skew = jnp.stack([
        jnp.stack([z, -t2, t1], axis=-1),
        jnp.stack([t2, z, -t0], axis=-1),
        jnp.stack([-t1, t0, z], axis=-1)], axis=1)
    F = jnp.transpose(jnp.linalg.inv(K1s), (0, 2, 1)) @ skew @ R @ jnp.linalg.inv(K0s)
    xs = jnp.arange(_GW, dtype=jnp.float32)
    ys = jnp.arange(_GH, dtype=jnp.float32)
    gx, gy = jnp.meshgrid(xs, ys, indexing='xy')
    coord = jnp.stack([gx, gy], axis=-1).reshape(_S, 2)
    p0 = jnp.concatenate([coord, jnp.ones((_S, 1), dtype=jnp.float32)], axis=-1)
    lines = jnp.einsum('nij,sj->nsi', F, p0).reshape(_S, 3)

    g1 = ln1_g.reshape(1, _D)
    b1 = ln1_b.reshape(1, _D)
    g2 = ln2_g.reshape(1, _D)
    b2 = ln2_b.reshape(1, _D)
    x2 = x.reshape(_S, _D)
    s2 = source.reshape(_S, _D)

    out = pl.pallas_call(
        _attn_kernel,
        grid=(_NBLK,),
        in_specs=[
            pl.BlockSpec((_BQ, 3), lambda i: (i, 0)),
            pl.BlockSpec((_BQ, _D), lambda i: (i, 0)),
            pl.BlockSpec((_S, _D), lambda i: (0, 0)),
            pl.BlockSpec((_D, _D), lambda i: (0, 0)),
            pl.BlockSpec((_D, _D), lambda i: (0, 0)),
            pl.BlockSpec((_D, _D), lambda i: (0, 0)),
            pl.BlockSpec((_D, _D), lambda i: (0, 0)),
            pl.BlockSpec((2 * _D, 2 * _D), lambda i: (0, 0)),
            pl.BlockSpec((_D, 2 * _D), lambda i: (0, 0)),
            pl.BlockSpec((1, _D), lambda i: (0, 0)),
            pl.BlockSpec((1, _D), lambda i: (0, 0)),
            pl.BlockSpec((1, _D), lambda i: (0, 0)),
            pl.BlockSpec((1, _D), lambda i: (0, 0)),
        ],
        out_specs=pl.BlockSpec((_BQ, _D), lambda i: (i, 0)),
        out_shape=jax.ShapeDtypeStruct((_S, _D), jnp.float32),
        scratch_shapes=[
            pltpu.VMEM((_S, _D), jnp.float32),
            pltpu.VMEM((_S, _D), jnp.float32),
        ],
    )(lines, x2, s2, Wq, Wk, Wv, Wmerge, Wmlp1, Wmlp2, g1, b1, g2, b2)

    return out.reshape(_N, _S, _D)


# speedup vs baseline: 152.8188x; 1.2896x over previous
import jax
import jax.numpy as jnp
from jax.experimental import pallas as pl
from jax.experimental.pallas import tpu as pltpu

_N = 1
_GH = 32
_GW = 32
_S = _GH * _GW
_D = 256
_NHEAD = 8
_DIM = _D // _NHEAD
_AW = 5
_C = max(_GH, _GW) * _AW
_SCALE = 16
_BQ = 1024
_NBLK = _S // _BQ

_NT = (((1,), (1,)), ((), ()))


def _perm_matrix():
    row = jax.lax.broadcasted_iota(jnp.int32, (_D, _D), 0)
    col = jax.lax.broadcasted_iota(jnp.int32, (_D, _D), 1)
    tgt = (col % _DIM) * _NHEAD + col // _DIM
    return (row == tgt).astype(jnp.float32)


def _attn_kernel(ln_ref, x_ref, s_ref, wq_ref, wk_ref, wv_ref, wm_ref, w1_ref,
                 w2_ref, g1_ref, b1_ref, g2_ref, b2_ref, o_ref, k_s, v_s):
    f32 = jnp.float32
    p = _perm_matrix()

    @pl.when(pl.program_id(0) == 0)
    def _project_kv():
        s = s_ref[...]
        kr = jax.lax.dot_general(s, wk_ref[...], _NT, preferred_element_type=f32)
        vr = jax.lax.dot_general(s, wv_ref[...], _NT, preferred_element_type=f32)
        k_s[...] = jnp.dot(kr, p, preferred_element_type=f32)
        v_s[...] = jnp.dot(vr, p, preferred_element_type=f32)

    lines = ln_ref[...]
    l0 = lines[:, 0:1]
    l1 = lines[:, 1:2]
    l2 = lines[:, 2:3]

    cols = jax.lax.broadcasted_iota(jnp.int32, (1, _S), 1)
    sx = (cols % _GW).astype(jnp.float32)
    sy = (cols // _GW).astype(jnp.float32)
    half = jnp.float32(_AW // 2)
    r1 = jnp.float32(1.0) / l1
    r0 = jnp.float32(1.0) / l0
    dy = sy + (l0 * sx + l2) * r1
    dx = sx + (l1 * sy + l2) * r0
    wy = jnp.abs(dy) < half
    wx = jnp.abs(dx) < half
    mode = jnp.abs(l1) > jnp.abs(l0)
    within = (mode & wy) | (jnp.logical_not(mode) & wx)
    cnt = jnp.sum(within.astype(jnp.int32), axis=1, keepdims=True)
    novalid = cnt == 0

    xb = x_ref[...]
    qr = jax.lax.dot_general(xb, wq_ref[...], _NT, preferred_element_type=f32)
    scale = jnp.float32(1.0 / (_DIM ** 0.5))
    q = jnp.dot(qr, p, preferred_element_type=f32) * scale
    k = k_s[...]
    v = v_s[...]
    v_fb = jnp.sum(v[:_C, :], axis=0, keepdims=True) * jnp.float32(1.0 / _C)
    msg_parts = []
    for h in range(_NHEAD):
        qh = q[:, h * _DIM:(h + 1) * _DIM]
        kh = k[:, h * _DIM:(h + 1) * _DIM]
        vh = v[:, h * _DIM:(h + 1) * _DIM]
        lg = jax.lax.dot_general(qh, kh, _NT, preferred_element_type=f32)
        e = jnp.where(within, jnp.exp(lg), jnp.float32(0.0))
        denom = jnp.sum(e, axis=1, keepdims=True)
        denom = jnp.where(novalid, jnp.float32(1.0), denom)
        mh = jnp.dot(e, vh, preferred_element_type=f32) / denom
        mh = jnp.where(novalid, v_fb[:, h * _DIM:(h + 1) * _DIM], mh)
        msg_parts.append(mh)
    msg = jnp.concatenate(msg_parts, axis=1)

    pt = jnp.transpose(_perm_matrix())
    msg_ref_order = jnp.dot(msg, pt, preferred_element_type=f32)
    merged = jax.lax.dot_general(msg_ref_order, wm_ref[...], _NT,
                                 preferred_element_type=f32)
    mu = jnp.mean(merged, axis=1, keepdims=True)
    var = jnp.mean((merged - mu) ** 2, axis=1, keepdims=True)
    msgn = (merged - mu) / jnp.sqrt(var + 1e-5) * g1_ref[...] + b1_ref[...]

    w1 = w1_ref[...]
    h1 = (jax.lax.dot_general(xb, w1[:, :_D], _NT, preferred_element_type=f32)
          + jax.lax.dot_general(msgn, w1[:, _D:], _NT, preferred_element_type=f32))
    h1 = jnp.maximum(h1, jnp.float32(0.0))
    h2 = jax.lax.dot_general(h1, w2_ref[...], _NT, preferred_element_type=f32)
    mu2 = jnp.mean(h2, axis=1, keepdims=True)
    var2 = jnp.mean((h2 - mu2) ** 2, axis=1, keepdims=True)
    h2n = (h2 - mu2) / jnp.sqrt(var2 + 1e-5) * g2_ref[...] + b2_ref[...]
    o_ref[...] = xb + h2n


def kernel(x, source, K0, K1, R, t, Wq, Wk, Wv, Wmerge, Wmlp1, Wmlp2,
           ln1_g, ln1_b, ln2_g, ln2_b):
    K0s = jnp.concatenate([K0[:, :2, :] / _SCALE, K0[:, 2:, :]], axis=1)
    K1s = jnp.concatenate([K1[:, :2, :] / _SCALE, K1[:, 2:, :]], axis=1)
    tv = t[:, :, 0]
    t0, t1, t2 = tv[:, 0], tv[:, 1], tv[:, 2]
    z = jnp.zeros_like(t0)
    skew = jnp.stack([
        jnp.stack([z, -t2, t1], axis=-1),
        jnp.stack([t2, z, -t0], axis=-1),
        jnp.stack([-t1, t0, z], axis=-1)], axis=1)
    F = jnp.transpose(jnp.linalg.inv(K1s), (0, 2, 1)) @ skew @ R @ jnp.linalg.inv(K0s)
    xs = jnp.arange(_GW, dtype=jnp.float32)
    ys = jnp.arange(_GH, dtype=jnp.float32)
    gx, gy = jnp.meshgrid(xs, ys, indexing='xy')
    coord = jnp.stack([gx, gy], axis=-1).reshape(_S, 2)
    p0 = jnp.concatenate([coord, jnp.ones((_S, 1), dtype=jnp.float32)], axis=-1)
    lines = jnp.einsum('nij,sj->nsi', F, p0).reshape(_S, 3)

    g1 = ln1_g.reshape(1, _D)
    b1 = ln1_b.reshape(1, _D)
    g2 = ln2_g.reshape(1, _D)
    b2 = ln2_b.reshape(1, _D)
    x2 = x.reshape(_S, _D)
    s2 = source.reshape(_S, _D)

    out = pl.pallas_call(
        _attn_kernel,
        grid=(_NBLK,),
        in_specs=[
            pl.BlockSpec((_BQ, 3), lambda i: (i, 0)),
            pl.BlockSpec((_BQ, _D), lambda i: (i, 0)),
            pl.BlockSpec((_S, _D), lambda i: (0, 0)),
            pl.BlockSpec((_D, _D), lambda i: (0, 0)),
            pl.BlockSpec((_D, _D), lambda i: (0, 0)),
            pl.BlockSpec((_D, _D), lambda i: (0, 0)),
            pl.BlockSpec((_D, _D), lambda i: (0, 0)),
            pl.BlockSpec((2 * _D, 2 * _D), lambda i: (0, 0)),
            pl.BlockSpec((_D, 2 * _D), lambda i: (0, 0)),
            pl.BlockSpec((1, _D), lambda i: (0, 0)),
            pl.BlockSpec((1, _D), lambda i: (0, 0)),
            pl.BlockSpec((1, _D), lambda i: (0, 0)),
            pl.BlockSpec((1, _D), lambda i: (0, 0)),
        ],
        out_specs=pl.BlockSpec((_BQ, _D), lambda i: (i, 0)),
        out_shape=jax.ShapeDtypeStruct((_S, _D), jnp.float32),
        scratch_shapes=[
            pltpu.VMEM((_S, _D), jnp.float32),
            pltpu.VMEM((_S, _D), jnp.float32),
        ],
    )(lines, x2, s2, Wq, Wk, Wv, Wmerge, Wmlp1, Wmlp2, g1, b1, g2, b2)

    return out.reshape(_N, _S, _D)
